# baseline (speedup 1.0000x reference)
import functools

import jax
import jax.numpy as jnp
from jax import lax
from jax.experimental import pallas as pl
from jax.experimental.pallas import tpu as pltpu
from jax.experimental.pallas import tpu_sc as plsc

EMB = 256
NG = 64
NNODE = 10000
CH = 32


def _cdiv(a, b):
    return (a + b - 1) // b


def _mesh():
    return plsc.VectorSubcoreMesh(core_axis_name="c", subcore_axis_name="s",
                                  num_cores=2, num_subcores=16)


def _zero_rows(buf, nrows, ncol16):
    z = jnp.zeros((16,), jnp.float32)

    def zb(r, carry):
        for k in range(ncol16):
            buf[r, pl.ds(k * 16, 16)] = z
        return carry

    lax.fori_loop(0, nrows, zb, 0)


def _spread_copy(src_fn, dst_fn, s, n_rows, unit):
    units = n_rows // unit

    def kbody(k, carry):
        u = k * 16 + s
        pltpu.sync_copy(src_fn(u * unit, unit), dst_fn(u * unit, unit))
        return carry

    trip = lax.max(0, (units - s + 15) // 16)
    lax.fori_loop(0, trip, kbody, 0)


@functools.lru_cache(maxsize=None)
def _msg_stage(E, base_a, stride_a, base_b, stride_b, with_sq, emit_rel):
    nch = E // CH

    out_type = [jax.ShapeDtypeStruct((2 * E, 128), jnp.float32)]
    if emit_rel:
        out_type.append(jax.ShapeDtypeStruct((E, 128), jnp.float32))

    scratch = dict(
        ia=pltpu.VMEM((CH,), jnp.int32),
        ib=pltpu.VMEM((CH,), jnp.int32),
        iaa=pltpu.VMEM((CH,), jnp.int32),
        iba=pltpu.VMEM((CH,), jnp.int32),
        ga=pltpu.VMEM((CH, 128), jnp.float32),
        gb=pltpu.VMEM((CH, 128), jnp.float32),
        sem0=pltpu.SemaphoreType.DMA,
        sem1=pltpu.SemaphoreType.DMA,
    )
    if with_sq:
        scratch.update(
            sqv=pltpu.VMEM((CH, 128), jnp.float32),
            w1cv=pltpu.VMEM((1024,), jnp.float32),
            relb=pltpu.VMEM((CH, 128), jnp.float32),
        )

    def body(ta, tb, ia_hbm, ib_hbm, *rest, **scr):
        if with_sq:
            x128, w1c_hbm = rest[0], rest[1]
            rest = rest[2:]
        h_hbm = rest[0]
        if emit_rel:
            rel_hbm = rest[1]
        c = lax.axis_index("c")
        s = lax.axis_index("s")
        ia, ib, iaa, iba, ga, gb = (
            scr["ia"], scr["ib"], scr["iaa"], scr["iba"],
            scr["ga"], scr["gb"])
        sem0, sem1 = scr["sem0"], scr["sem1"]

        if with_sq:
            pltpu.sync_copy(w1c_hbm.at[pl.ds(c * 1024, 1024)], scr["w1cv"])

        off_a = base_a + c * stride_a
        off_b = base_b + c * stride_b

        def chunk(g):
            base = g * CH
            pltpu.sync_copy(ia_hbm.at[pl.ds(base, CH)], ia)
            pltpu.sync_copy(ib_hbm.at[pl.ds(base, CH)], ib)
            for v in range(CH // 16):
                sl = pl.ds(v * 16, 16)
                iaa[sl] = ia[sl] + off_a
                iba[sl] = ib[sl] + off_b
            if with_sq:
                dxd = pltpu.async_copy(x128.at[ia], ga, sem0)
                dxs = pltpu.async_copy(x128.at[ib], gb, sem1)
                dxd.wait()
                dxs.wait()

                def xbody(e, carry):
                    rv0 = ga[e, pl.ds(0, 16)] - gb[e, pl.ds(0, 16)]
                    rv1 = ga[e, pl.ds(16, 16)] - gb[e, pl.ds(16, 16)]
                    rv2 = ga[e, pl.ds(32, 16)] - gb[e, pl.ds(32, 16)]
                    scr["relb"][e, pl.ds(0, 16)] = rv0
                    scr["relb"][e, pl.ds(16, 16)] = rv1
                    scr["relb"][e, pl.ds(32, 16)] = rv2
                    scr["sqv"][e, pl.ds(0, 16)] = rv0 * rv0 + rv1 * rv1 + rv2 * rv2
                    return carry

                lax.fori_loop(0, CH, xbody, 0)
            da = pltpu.async_copy(ta.at[iaa], ga, sem0)
            db = pltpu.async_copy(tb.at[iba], gb, sem1)
            da.wait()
            db.wait()

            def ebody(e, carry):
                if with_sq:
                    sv = scr["sqv"][e, pl.ds(0, 16)]
                for k in range(8):
                    sl = pl.ds(k * 16, 16)
                    z = ga[e, sl] + gb[e, sl]
                    if with_sq:
                        z = z + sv * scr["w1cv"][pl.ds(k * 16, 16)]
                    ga[e, sl] = jnp.maximum(z, 0.0)
                return carry

            lax.fori_loop(0, CH, ebody, 0)
            pltpu.sync_copy(ga, h_hbm.at[pl.ds(c * E + base, CH)])
            if emit_rel:
                pltpu.sync_copy(scr["relb"], rel_hbm.at[pl.ds(base, CH)])

        def kbody(k, carry):
            chunk(k * 16 + s)
            return carry

        trip = lax.max(0, (nch - s + 15) // 16)
        lax.fori_loop(0, trip, kbody, 0)

    def body_flat(*args):
        n_in = 4 + (2 if with_sq else 0)
        n_out = len(out_type)
        ins = args[:n_in]
        outs = args[n_in:n_in + n_out]
        scrs = args[n_in + n_out:]
        body(*ins, *outs, **dict(zip(scratch.keys(), scrs)))

    ot = out_type if emit_rel else out_type[0]
    return pl.kernel(body_flat, out_type=ot, mesh=_mesh(),
                     scratch_types=list(scratch.values()))


def _mm_body(a_ref, w_ref, b_ref, o_ref, *, relu):
    acc = jnp.dot(a_ref[...], w_ref[...], preferred_element_type=jnp.float32)
    acc = acc + b_ref[...]
    if relu:
        acc = jnp.maximum(acc, 0.0)
    o_ref[...] = acc


def _mm(a, w, b=None, relu=False, bm=512):
    m, k = a.shape
    n = w.shape[1]
    if b is None:
        b = jnp.zeros((n,), jnp.float32)
    grid = (_cdiv(m, bm),)
    return pl.pallas_call(
        functools.partial(_mm_body, relu=relu),
        grid=grid,
        in_specs=[
            pl.BlockSpec((bm, k), lambda i: (i, 0)),
            pl.BlockSpec((k, n), lambda i: (0, 0)),
            pl.BlockSpec((1, n), lambda i: (0, 0)),
        ],
        out_specs=pl.BlockSpec((bm, n), lambda i: (i, 0)),
        out_shape=jax.ShapeDtypeStruct((m, n), jnp.float32),
    )(a, w, b.reshape(1, n))


def _mm_split_body(a_ref, w_ref, b_ref, o_ref):
    o_ref[0] = (jnp.dot(a_ref[...], w_ref[0], preferred_element_type=jnp.float32)
                + b_ref[0])


def _mm_split(a, w3, b2, bm=512):
    m, k = a.shape
    jd = w3.shape[0]
    b2 = b2.reshape(jd, 1, 128)
    grid = (_cdiv(m, bm), jd)
    out = pl.pallas_call(
        _mm_split_body,
        grid=grid,
        in_specs=[
            pl.BlockSpec((bm, k), lambda i, j: (i, 0)),
            pl.BlockSpec((1, k, 128), lambda i, j: (j, 0, 0)),
            pl.BlockSpec((1, 1, 128), lambda i, j: (j, 0, 0)),
        ],
        out_specs=pl.BlockSpec((1, bm, 128), lambda i, j: (j, i, 0)),
        out_shape=jax.ShapeDtypeStruct((jd, m, 128), jnp.float32),
    )(a, w3, b2)
    return out.reshape(jd * m, 128)


def _coef_body(ha_ref, hb_ref, qa_ref, qb_ref, q0_ref, p2_ref, pb2_ref, o_ref):
    t = (jnp.dot(ha_ref[...], qa_ref[...], preferred_element_type=jnp.float32)
         + jnp.dot(hb_ref[...], qb_ref[...], preferred_element_type=jnp.float32)
         + q0_ref[...])
    t = jnp.maximum(t, 0.0)
    res = jnp.dot(t, p2_ref[...], preferred_element_type=jnp.float32) + pb2_ref[...]
    o_ref[...] = jnp.broadcast_to(res, (res.shape[0], 128))


def _coef_mlp(ha, hb, qa, qb, q0, p2, pb2, bm=1024):
    m = ha.shape[0]
    grid = (_cdiv(m, bm),)
    return pl.pallas_call(
        _coef_body,
        grid=grid,
        in_specs=[
            pl.BlockSpec((bm, 128), lambda i: (i, 0)),
            pl.BlockSpec((bm, 128), lambda i: (i, 0)),
            pl.BlockSpec((128, EMB), lambda i: (0, 0)),
            pl.BlockSpec((128, EMB), lambda i: (0, 0)),
            pl.BlockSpec((1, EMB), lambda i: (0, 0)),
            pl.BlockSpec((EMB, 1), lambda i: (0, 0)),
            pl.BlockSpec((1, 1), lambda i: (0, 0)),
        ],
        out_specs=pl.BlockSpec((bm, 128), lambda i: (i, 0)),
        out_shape=jax.ShapeDtypeStruct((m, 128), jnp.float32),
    )(ha, hb, qa, qb, q0.reshape(1, EMB), p2, pb2.reshape(1, 1))


def _upd_body(h_ref, s1a, s1b, s2a, s2b, sea, seb, ssa, ssb, d_ref, w_ref,
              ub1_ref, u2_ref, ub2_ref, o_ref, o2_ref):
    E = EMB

    def dt(x_ref, lo, hi):
        return jnp.dot(x_ref[...], w_ref[lo:hi, :], preferred_element_type=jnp.float32)

    z = (dt(h_ref, 0, E) + dt(s1a, E, E + 128) + dt(s1b, E + 128, 2 * E)
         + dt(s2a, 2 * E, 2 * E + 128) + dt(s2b, 2 * E + 128, 3 * E)
         + dt(sea, 3 * E, 3 * E + 128) + dt(seb, 3 * E + 128, 4 * E)
         + dt(ssa, 4 * E, 4 * E + 128) + dt(ssb, 4 * E + 128, 5 * E)
         + dt(d_ref, 5 * E, 5 * E + 8) + ub1_ref[...])
    z = jnp.maximum(z, 0.0)
    out = (h_ref[...] + jnp.dot(z, u2_ref[...], preferred_element_type=jnp.float32)
           + ub2_ref[...])
    o_ref[...] = out
    o2_ref[0] = out[:, :128]
    o2_ref[1] = out[:, 128:]


def _upd_mlp(h0, s1f, s2f, sef, ssf, d8, wcomb, ub1, u2, ub2, bm=400):
    m = h0.shape[0]
    nblk = m // bm
    grid = (nblk,)
    spec_a = pl.BlockSpec((bm, 128), lambda i: (i, 0))
    spec_b = pl.BlockSpec((bm, 128), lambda i: (nblk + i, 0))
    return pl.pallas_call(
        _upd_body,
        grid=grid,
        in_specs=[
            pl.BlockSpec((bm, EMB), lambda i: (i, 0)),
            spec_a, spec_b, spec_a, spec_b, spec_a, spec_b, spec_a, spec_b,
            pl.BlockSpec((bm, 8), lambda i: (i, 0)),
            pl.BlockSpec((5 * EMB + 8, EMB), lambda i: (0, 0)),
            pl.BlockSpec((1, EMB), lambda i: (0, 0)),
            pl.BlockSpec((EMB, EMB), lambda i: (0, 0)),
            pl.BlockSpec((1, EMB), lambda i: (0, 0)),
        ],
        out_specs=[
            pl.BlockSpec((bm, EMB), lambda i: (i, 0)),
            pl.BlockSpec((2, bm, 128), lambda i: (0, i, 0)),
        ],
        out_shape=[
            jax.ShapeDtypeStruct((m, EMB), jnp.float32),
            jax.ShapeDtypeStruct((2, m, 128), jnp.float32),
        ],
    )(h0, s1f, s1f, s2f, s2f, sef, sef, ssf, ssf, d8, wcomb,
      ub1.reshape(1, EMB), u2, ub2.reshape(1, EMB))


def _split_w(w, k):
    n = w.shape[1]
    return jnp.transpose(w.reshape(k, n // 128, 128), (1, 0, 2))


def kernel(x, pos, edge_attr, sse_attr, batch_idx, adj1_index, adj2_index,
           n1_edge, n1_node, n2_sse, n2_node, params):
    N = x.shape[0]
    E = EMB
    d1, s1 = adj1_index[0], adj1_index[1]
    d2, s2 = adj2_index[0], adj2_index[1]
    E1 = d1.shape[0]
    EN1 = n1_node.shape[0]
    EN2 = n2_node.shape[0]
    NS = sse_attr.shape[0]

    H0 = _mm(x, params["emb0_w"], params["emb0_b"])
    X = pos

    deg1 = jax.ops.segment_sum(jnp.ones(d1.shape, jnp.float32), d1, num_segments=N)
    deg2 = jax.ops.segment_sum(jnp.ones(d2.shape, jnp.float32), d2, num_segments=N)
    dege = jax.ops.segment_sum(jnp.ones(n1_node.shape, jnp.float32), n1_node, num_segments=N)
    degs = jax.ops.segment_sum(jnp.ones(n2_node.shape, jnp.float32), n2_node, num_segments=N)
    countb = jax.ops.segment_sum(jnp.ones((N,), jnp.float32), batch_idx, num_segments=NG)
    zcol = jnp.zeros((N,), jnp.float32)
    d8 = jnp.stack([deg1, deg2, dege, degs, zcol, zcol, zcol, zcol], axis=1)

    msg1 = _msg_stage(E1, 0 * N, N, 2 * N, N, True, True)
    msg2 = _msg_stage(E1, 4 * N, N, 6 * N, N, True, False)
    msge = _msg_stage(EN1, 8 * N, N, 0, E1, False, False)
    msgs = _msg_stage(EN2, 10 * N, N, 0, NS, False, False)
    d1_2 = jnp.concatenate([d1, d1 + N])
    d2_2 = jnp.concatenate([d2, d2 + N])
    ne_2 = jnp.concatenate([n1_node, n1_node + N])
    ns_2 = jnp.concatenate([n2_node, n2_node + N])

    H0split = None
    for lp in params["layers"]:
        W1_1, b1_1, W2_1, b2_1 = lp["msg1"]
        W1_2, b1_2, W2_2, b2_2 = lp["msg2"]
        W1_e, b1_e, W2_e, b2_e = lp["msge"]
        W1_s, b1_s, W2_s, b2_s = lp["msgs"]
        U1, ub1, U2, ub2 = lp["upd"]
        P1, pb1, P2, pb2 = lp["posm"]

        wbig = jnp.concatenate(
            [W1_1[:E], W1_1[E:2 * E], W1_2[:E], W1_2[E:2 * E], W1_e[:E], W1_s[:E]], axis=1)
        bbig = jnp.concatenate(
            [b1_1, jnp.zeros((E,), jnp.float32), b1_2] + [jnp.zeros((E,), jnp.float32)] * 3)
        pflat = _mm_split(H0, _split_w(wbig, E), bbig.reshape(12, 128))
        eflat = _mm_split(edge_attr, _split_w(W1_e[E:], 2), b1_e.reshape(2, 128), bm=2048)
        sflat = _mm_split(sse_attr, _split_w(W1_s[E:], 4), b1_s.reshape(2, 128), bm=2000)

        x128 = jnp.pad(jnp.repeat(X, 16, axis=1), ((0, 0), (0, 80)))
        w1c_1 = jnp.pad(W1_1[2 * E].reshape(2, 128), ((0, 0), (0, 896))).reshape(2048)
        w1c_2 = jnp.pad(W1_2[2 * E].reshape(2, 128), ((0, 0), (0, 896))).reshape(2048)

        h1f, rel1 = msg1(pflat, pflat, d1, s1, x128, w1c_1)
        h2f = msg2(pflat, pflat, d2, s2, x128, w1c_2)
        hef = msge(pflat, eflat, n1_node, n1_edge)
        hsf = msgs(pflat, sflat, n2_node, n2_sse)
        s1f = jax.ops.segment_sum(h1f, d1_2, num_segments=2 * N)
        s2f = jax.ops.segment_sum(h2f, d2_2, num_segments=2 * N)
        sef = jax.ops.segment_sum(hef, ne_2, num_segments=2 * N)
        ssf = jax.ops.segment_sum(hsf, ns_2, num_segments=2 * N)

        wcomb = jnp.concatenate(
            [U1[:E], W2_1 @ U1[E:2 * E], W2_2 @ U1[2 * E:3 * E],
             W2_e @ U1[3 * E:4 * E], W2_s @ U1[4 * E:],
             jnp.stack([b2_1 @ U1[E:2 * E], b2_2 @ U1[2 * E:3 * E],
                        b2_e @ U1[3 * E:4 * E], b2_s @ U1[4 * E:]]
                       + [jnp.zeros((E,), jnp.float32)] * 4)], axis=0)
        H0, H0split = _upd_mlp(H0, s1f, s2f, sef, ssf, d8, wcomb, ub1, U2, ub2)

        Q = W2_1 @ P1
        q0 = b2_1 @ P1 + pb1
        coef = _coef_mlp(h1f[:E1], h1f[E1:], Q[:128], Q[128:], q0, P2, pb2)
        rel3 = jnp.stack([rel1[:, 0], rel1[:, 16], rel1[:, 32]], axis=1)
        agx = jax.ops.segment_sum(rel3 * coef[:, :1], d1, num_segments=N)
        X = X + agx / (deg1 + 1.0)[:, None]

    gsum = jax.ops.segment_sum(H0, batch_idx, num_segments=NG)
    graph_emb = gsum / jnp.maximum(countb, 1.0)[:, None]
    return (H0, graph_emb, X)


# speedup vs baseline: 1.2368x; 1.2368x over previous
import functools

import jax
import jax.numpy as jnp
from jax import lax
from jax.experimental import pallas as pl
from jax.experimental.pallas import tpu as pltpu
from jax.experimental.pallas import tpu_sc as plsc

EMB = 256
NG = 64
NNODE = 10000
CH = 32


def _cdiv(a, b):
    return (a + b - 1) // b


def _mesh():
    return plsc.VectorSubcoreMesh(core_axis_name="c", subcore_axis_name="s",
                                  num_cores=2, num_subcores=16)


def _zero_rows(buf, nrows, ncol16):
    z = jnp.zeros((16,), jnp.float32)

    def zb(r, carry):
        for k in range(ncol16):
            buf[r, pl.ds(k * 16, 16)] = z
        return carry

    lax.fori_loop(0, nrows, zb, 0)


def _spread_copy(src_fn, dst_fn, s, n_rows, unit):
    units = n_rows // unit

    def kbody(k, carry):
        u = k * 16 + s
        pltpu.sync_copy(src_fn(u * unit, unit), dst_fn(u * unit, unit))
        return carry

    trip = lax.max(0, (units - s + 15) // 16)
    lax.fori_loop(0, trip, kbody, 0)


@functools.lru_cache(maxsize=None)
def _msg_stage(E, base_a, stride_a, base_b, stride_b, with_sq, emit_rel):
    ch = 128 if E % 128 == 0 else 64
    nch = E // ch

    out_type = [jax.ShapeDtypeStruct((2 * E, 128), jnp.float32)]
    if emit_rel:
        out_type.append(jax.ShapeDtypeStruct((E, 128), jnp.float32))

    scratch = dict(
        ia=pltpu.VMEM((ch,), jnp.int32),
        ib=pltpu.VMEM((ch,), jnp.int32),
        iaa=pltpu.VMEM((ch,), jnp.int32),
        iba=pltpu.VMEM((ch,), jnp.int32),
        ga=pltpu.VMEM((ch, 128), jnp.float32),
        gb=pltpu.VMEM((ch, 128), jnp.float32),
        sem0=pltpu.SemaphoreType.DMA,
        sem1=pltpu.SemaphoreType.DMA,
    )
    if with_sq:
        scratch.update(
            sqv=pltpu.VMEM((ch, 128), jnp.float32),
            w1cv=pltpu.VMEM((1024,), jnp.float32),
            relb=pltpu.VMEM((ch, 128), jnp.float32),
        )

    def body(ta, tb, ia_hbm, ib_hbm, *rest, **scr):
        if with_sq:
            x128, w1c_hbm = rest[0], rest[1]
            rest = rest[2:]
        h_hbm = rest[0]
        if emit_rel:
            rel_hbm = rest[1]
        c = lax.axis_index("c")
        s = lax.axis_index("s")
        ia, ib, iaa, iba, ga, gb = (
            scr["ia"], scr["ib"], scr["iaa"], scr["iba"],
            scr["ga"], scr["gb"])
        sem0, sem1 = scr["sem0"], scr["sem1"]

        if with_sq:
            pltpu.sync_copy(w1c_hbm.at[pl.ds(c * 1024, 1024)], scr["w1cv"])

        off_a = base_a + c * stride_a
        off_b = base_b + c * stride_b

        def chunk(g):
            base = g * ch
            pltpu.sync_copy(ia_hbm.at[pl.ds(base, ch)], ia)
            pltpu.sync_copy(ib_hbm.at[pl.ds(base, ch)], ib)
            for v in range(ch // 16):
                sl = pl.ds(v * 16, 16)
                iaa[sl] = ia[sl] + off_a
                iba[sl] = ib[sl] + off_b
            if with_sq:
                dxd = pltpu.async_copy(x128.at[ia], ga, sem0)
                dxs = pltpu.async_copy(x128.at[ib], gb, sem1)
                dxd.wait()
                dxs.wait()

                def xbody(e, carry):
                    rv0 = ga[e, pl.ds(0, 16)] - gb[e, pl.ds(0, 16)]
                    rv1 = ga[e, pl.ds(16, 16)] - gb[e, pl.ds(16, 16)]
                    rv2 = ga[e, pl.ds(32, 16)] - gb[e, pl.ds(32, 16)]
                    scr["relb"][e, pl.ds(0, 16)] = rv0
                    scr["relb"][e, pl.ds(16, 16)] = rv1
                    scr["relb"][e, pl.ds(32, 16)] = rv2
                    scr["sqv"][e, pl.ds(0, 16)] = rv0 * rv0 + rv1 * rv1 + rv2 * rv2
                    return carry

                lax.fori_loop(0, ch, xbody, 0)
            da = pltpu.async_copy(ta.at[iaa], ga, sem0)
            db = pltpu.async_copy(tb.at[iba], gb, sem1)
            da.wait()
            db.wait()

            def ebody(e, carry):
                if with_sq:
                    sv = scr["sqv"][e, pl.ds(0, 16)]
                for k in range(8):
                    sl = pl.ds(k * 16, 16)
                    z = ga[e, sl] + gb[e, sl]
                    if with_sq:
                        z = z + sv * scr["w1cv"][pl.ds(k * 16, 16)]
                    ga[e, sl] = jnp.maximum(z, 0.0)
                return carry

            lax.fori_loop(0, ch, ebody, 0)
            pltpu.sync_copy(ga, h_hbm.at[pl.ds(c * E + base, ch)])
            if emit_rel:
                pltpu.sync_copy(scr["relb"], rel_hbm.at[pl.ds(base, ch)])

        def kbody(k, carry):
            chunk(k * 16 + s)
            return carry

        trip = lax.max(0, (nch - s + 15) // 16)
        lax.fori_loop(0, trip, kbody, 0)

    def body_flat(*args):
        n_in = 4 + (2 if with_sq else 0)
        n_out = len(out_type)
        ins = args[:n_in]
        outs = args[n_in:n_in + n_out]
        scrs = args[n_in + n_out:]
        body(*ins, *outs, **dict(zip(scratch.keys(), scrs)))

    ot = out_type if emit_rel else out_type[0]
    return pl.kernel(body_flat, out_type=ot, mesh=_mesh(),
                     scratch_types=list(scratch.values()))


def _mm_body(a_ref, w_ref, b_ref, o_ref, *, relu):
    acc = jnp.dot(a_ref[...], w_ref[...], preferred_element_type=jnp.float32)
    acc = acc + b_ref[...]
    if relu:
        acc = jnp.maximum(acc, 0.0)
    o_ref[...] = acc


def _mm(a, w, b=None, relu=False, bm=512):
    m, k = a.shape
    n = w.shape[1]
    if b is None:
        b = jnp.zeros((n,), jnp.float32)
    grid = (_cdiv(m, bm),)
    return pl.pallas_call(
        functools.partial(_mm_body, relu=relu),
        grid=grid,
        in_specs=[
            pl.BlockSpec((bm, k), lambda i: (i, 0)),
            pl.BlockSpec((k, n), lambda i: (0, 0)),
            pl.BlockSpec((1, n), lambda i: (0, 0)),
        ],
        out_specs=pl.BlockSpec((bm, n), lambda i: (i, 0)),
        out_shape=jax.ShapeDtypeStruct((m, n), jnp.float32),
    )(a, w, b.reshape(1, n))


def _mm_split_body(a_ref, w_ref, b_ref, o_ref):
    o_ref[0] = (jnp.dot(a_ref[...], w_ref[0], preferred_element_type=jnp.float32)
                + b_ref[0])


def _mm_split(a, w3, b2, bm=512):
    m, k = a.shape
    jd = w3.shape[0]
    b2 = b2.reshape(jd, 1, 128)
    grid = (_cdiv(m, bm), jd)
    out = pl.pallas_call(
        _mm_split_body,
        grid=grid,
        in_specs=[
            pl.BlockSpec((bm, k), lambda i, j: (i, 0)),
            pl.BlockSpec((1, k, 128), lambda i, j: (j, 0, 0)),
            pl.BlockSpec((1, 1, 128), lambda i, j: (j, 0, 0)),
        ],
        out_specs=pl.BlockSpec((1, bm, 128), lambda i, j: (j, i, 0)),
        out_shape=jax.ShapeDtypeStruct((jd, m, 128), jnp.float32),
    )(a, w3, b2)
    return out.reshape(jd * m, 128)


def _coef_body(ha_ref, hb_ref, qa_ref, qb_ref, q0_ref, p2_ref, pb2_ref, o_ref):
    t = (jnp.dot(ha_ref[...], qa_ref[...], preferred_element_type=jnp.float32)
         + jnp.dot(hb_ref[...], qb_ref[...], preferred_element_type=jnp.float32)
         + q0_ref[...])
    t = jnp.maximum(t, 0.0)
    res = jnp.dot(t, p2_ref[...], preferred_element_type=jnp.float32) + pb2_ref[...]
    o_ref[...] = jnp.broadcast_to(res, (res.shape[0], 128))


def _coef_mlp(ha, hb, qa, qb, q0, p2, pb2, bm=1024):
    m = ha.shape[0]
    grid = (_cdiv(m, bm),)
    return pl.pallas_call(
        _coef_body,
        grid=grid,
        in_specs=[
            pl.BlockSpec((bm, 128), lambda i: (i, 0)),
            pl.BlockSpec((bm, 128), lambda i: (i, 0)),
            pl.BlockSpec((128, EMB), lambda i: (0, 0)),
            pl.BlockSpec((128, EMB), lambda i: (0, 0)),
            pl.BlockSpec((1, EMB), lambda i: (0, 0)),
            pl.BlockSpec((EMB, 1), lambda i: (0, 0)),
            pl.BlockSpec((1, 1), lambda i: (0, 0)),
        ],
        out_specs=pl.BlockSpec((bm, 128), lambda i: (i, 0)),
        out_shape=jax.ShapeDtypeStruct((m, 128), jnp.float32),
    )(ha, hb, qa, qb, q0.reshape(1, EMB), p2, pb2.reshape(1, 1))


def _upd_body(h_ref, s1a, s1b, s2a, s2b, sea, seb, ssa, ssb, d_ref, w_ref,
              ub1_ref, u2_ref, ub2_ref, o_ref, o2_ref):
    E = EMB

    def dt(x_ref, lo, hi):
        return jnp.dot(x_ref[...], w_ref[lo:hi, :], preferred_element_type=jnp.float32)

    z = (dt(h_ref, 0, E) + dt(s1a, E, E + 128) + dt(s1b, E + 128, 2 * E)
         + dt(s2a, 2 * E, 2 * E + 128) + dt(s2b, 2 * E + 128, 3 * E)
         + dt(sea, 3 * E, 3 * E + 128) + dt(seb, 3 * E + 128, 4 * E)
         + dt(ssa, 4 * E, 4 * E + 128) + dt(ssb, 4 * E + 128, 5 * E)
         + dt(d_ref, 5 * E, 5 * E + 8) + ub1_ref[...])
    z = jnp.maximum(z, 0.0)
    out = (h_ref[...] + jnp.dot(z, u2_ref[...], preferred_element_type=jnp.float32)
           + ub2_ref[...])
    o_ref[...] = out
    o2_ref[0] = out[:, :128]
    o2_ref[1] = out[:, 128:]


def _upd_mlp(h0, s1f, s2f, sef, ssf, d8, wcomb, ub1, u2, ub2, bm=400):
    m = h0.shape[0]
    nblk = m // bm
    grid = (nblk,)
    spec_a = pl.BlockSpec((bm, 128), lambda i: (i, 0))
    spec_b = pl.BlockSpec((bm, 128), lambda i: (nblk + i, 0))
    return pl.pallas_call(
        _upd_body,
        grid=grid,
        in_specs=[
            pl.BlockSpec((bm, EMB), lambda i: (i, 0)),
            spec_a, spec_b, spec_a, spec_b, spec_a, spec_b, spec_a, spec_b,
            pl.BlockSpec((bm, 8), lambda i: (i, 0)),
            pl.BlockSpec((5 * EMB + 8, EMB), lambda i: (0, 0)),
            pl.BlockSpec((1, EMB), lambda i: (0, 0)),
            pl.BlockSpec((EMB, EMB), lambda i: (0, 0)),
            pl.BlockSpec((1, EMB), lambda i: (0, 0)),
        ],
        out_specs=[
            pl.BlockSpec((bm, EMB), lambda i: (i, 0)),
            pl.BlockSpec((2, bm, 128), lambda i: (0, i, 0)),
        ],
        out_shape=[
            jax.ShapeDtypeStruct((m, EMB), jnp.float32),
            jax.ShapeDtypeStruct((2, m, 128), jnp.float32),
        ],
    )(h0, s1f, s1f, s2f, s2f, sef, sef, ssf, ssf, d8, wcomb,
      ub1.reshape(1, EMB), u2, ub2.reshape(1, EMB))


def _split_w(w, k):
    n = w.shape[1]
    return jnp.transpose(w.reshape(k, n // 128, 128), (1, 0, 2))


def kernel(x, pos, edge_attr, sse_attr, batch_idx, adj1_index, adj2_index,
           n1_edge, n1_node, n2_sse, n2_node, params):
    N = x.shape[0]
    E = EMB
    d1, s1 = adj1_index[0], adj1_index[1]
    d2, s2 = adj2_index[0], adj2_index[1]
    E1 = d1.shape[0]
    EN1 = n1_node.shape[0]
    EN2 = n2_node.shape[0]
    NS = sse_attr.shape[0]

    H0 = _mm(x, params["emb0_w"], params["emb0_b"])
    X = pos

    deg1 = jax.ops.segment_sum(jnp.ones(d1.shape, jnp.float32), d1, num_segments=N)
    deg2 = jax.ops.segment_sum(jnp.ones(d2.shape, jnp.float32), d2, num_segments=N)
    dege = jax.ops.segment_sum(jnp.ones(n1_node.shape, jnp.float32), n1_node, num_segments=N)
    degs = jax.ops.segment_sum(jnp.ones(n2_node.shape, jnp.float32), n2_node, num_segments=N)
    countb = jax.ops.segment_sum(jnp.ones((N,), jnp.float32), batch_idx, num_segments=NG)
    zcol = jnp.zeros((N,), jnp.float32)
    d8 = jnp.stack([deg1, deg2, dege, degs, zcol, zcol, zcol, zcol], axis=1)

    msg1 = _msg_stage(E1, 0 * N, N, 2 * N, N, True, True)
    msg2 = _msg_stage(E1, 4 * N, N, 6 * N, N, True, False)
    msge = _msg_stage(EN1, 8 * N, N, 0, E1, False, False)
    msgs = _msg_stage(EN2, 10 * N, N, 0, NS, False, False)
    d1_2 = jnp.concatenate([d1, d1 + N])
    d2_2 = jnp.concatenate([d2, d2 + N])
    ne_2 = jnp.concatenate([n1_node, n1_node + N])
    ns_2 = jnp.concatenate([n2_node, n2_node + N])

    H0split = None
    for lp in params["layers"]:
        W1_1, b1_1, W2_1, b2_1 = lp["msg1"]
        W1_2, b1_2, W2_2, b2_2 = lp["msg2"]
        W1_e, b1_e, W2_e, b2_e = lp["msge"]
        W1_s, b1_s, W2_s, b2_s = lp["msgs"]
        U1, ub1, U2, ub2 = lp["upd"]
        P1, pb1, P2, pb2 = lp["posm"]

        wbig = jnp.concatenate(
            [W1_1[:E], W1_1[E:2 * E], W1_2[:E], W1_2[E:2 * E], W1_e[:E], W1_s[:E]], axis=1)
        bbig = jnp.concatenate(
            [b1_1, jnp.zeros((E,), jnp.float32), b1_2] + [jnp.zeros((E,), jnp.float32)] * 3)
        pflat = _mm_split(H0, _split_w(wbig, E), bbig.reshape(12, 128))
        eflat = _mm_split(edge_attr, _split_w(W1_e[E:], 2), b1_e.reshape(2, 128), bm=2048)
        sflat = _mm_split(sse_attr, _split_w(W1_s[E:], 4), b1_s.reshape(2, 128), bm=2000)

        x128 = jnp.pad(jnp.repeat(X, 16, axis=1), ((0, 0), (0, 80)))
        w1c_1 = jnp.pad(W1_1[2 * E].reshape(2, 128), ((0, 0), (0, 896))).reshape(2048)
        w1c_2 = jnp.pad(W1_2[2 * E].reshape(2, 128), ((0, 0), (0, 896))).reshape(2048)

        h1f, rel1 = msg1(pflat, pflat, d1, s1, x128, w1c_1)
        h2f = msg2(pflat, pflat, d2, s2, x128, w1c_2)
        hef = msge(pflat, eflat, n1_node, n1_edge)
        hsf = msgs(pflat, sflat, n2_node, n2_sse)
        s1f = jax.ops.segment_sum(h1f, d1_2, num_segments=2 * N)
        s2f = jax.ops.segment_sum(h2f, d2_2, num_segments=2 * N)
        sef = jax.ops.segment_sum(hef, ne_2, num_segments=2 * N)
        ssf = jax.ops.segment_sum(hsf, ns_2, num_segments=2 * N)

        wcomb = jnp.concatenate(
            [U1[:E], W2_1 @ U1[E:2 * E], W2_2 @ U1[2 * E:3 * E],
             W2_e @ U1[3 * E:4 * E], W2_s @ U1[4 * E:],
             jnp.stack([b2_1 @ U1[E:2 * E], b2_2 @ U1[2 * E:3 * E],
                        b2_e @ U1[3 * E:4 * E], b2_s @ U1[4 * E:]]
                       + [jnp.zeros((E,), jnp.float32)] * 4)], axis=0)
        H0, H0split = _upd_mlp(H0, s1f, s2f, sef, ssf, d8, wcomb, ub1, U2, ub2)

        Q = W2_1 @ P1
        q0 = b2_1 @ P1 + pb1
        coef = _coef_mlp(h1f[:E1], h1f[E1:], Q[:128], Q[128:], q0, P2, pb2)
        rel3 = jnp.stack([rel1[:, 0], rel1[:, 16], rel1[:, 32]], axis=1)
        agx = jax.ops.segment_sum(rel3 * coef[:, :1], d1, num_segments=N)
        X = X + agx / (deg1 + 1.0)[:, None]

    gsum = jax.ops.segment_sum(H0, batch_idx, num_segments=NG)
    graph_emb = gsum / jnp.maximum(countb, 1.0)[:, None]
    return (H0, graph_emb, X)
